# baseline (speedup 1.0000x reference)
import functools

import jax
import jax.numpy as jnp
import numpy as np
from jax import lax
from jax.experimental import pallas as pl
from jax.experimental.pallas import tpu as pltpu
from jax.experimental.pallas import tpu_sc as plsc

_B, _F, _M = 32, 64, 128
_N = _B * _F
_T = _M * (_M - 1) // 2
_L = 16

_ROWS, _COLS = np.triu_indices(_M, k=1)
_IDX = np.asarray(_ROWS * _M + _COLS, dtype=np.int32)


@jax.jit
def _triu_sc(x2, idx):
    info = plsc.get_sparse_core_info()
    nw = info.num_cores * info.num_subcores
    npw = _N // nw

    mesh = plsc.VectorSubcoreMesh(core_axis_name="c", subcore_axis_name="s")

    @functools.partial(
        pl.kernel,
        mesh=mesh,
        out_type=jax.ShapeDtypeStruct((_N, _T), jnp.float32),
        scratch_types=[
            pltpu.VMEM((_M * _M,), jnp.float32),
            pltpu.VMEM((_T,), jnp.float32),
            pltpu.VMEM((_T,), jnp.int32),
        ],
        compiler_params=pltpu.CompilerParams(needs_layout_passes=False),
    )
    def k(x_hbm, idx_hbm, out_hbm, inb, outb, idxb):
        wid = lax.axis_index("s") * info.num_cores + lax.axis_index("c")
        pltpu.sync_copy(idx_hbm, idxb)

        def body(n, carry):
            pltpu.sync_copy(x_hbm.at[n], inb)

            def chunk(i, c):
                iv = idxb[pl.ds(i * _L, _L)]
                outb[pl.ds(i * _L, _L)] = plsc.load_gather(inb, [iv])
                return c

            lax.fori_loop(0, _T // _L, chunk, 0, unroll=8)
            pltpu.sync_copy(outb, out_hbm.at[n])
            return carry

        lax.fori_loop(wid * npw, (wid + 1) * npw, body, 0)

    return k(x2, idx)


def kernel(X):
    out = _triu_sc(X.reshape(_N, _M * _M), _IDX)
    return out.reshape(_B, _F, _T)


# speedup vs baseline: 1.2849x; 1.2849x over previous
import functools

import jax
import jax.numpy as jnp
import numpy as np
from jax import lax
from jax.experimental import pallas as pl
from jax.experimental.pallas import tpu as pltpu
from jax.experimental.pallas import tpu_sc as plsc

_B, _F, _M = 32, 64, 128
_N = _B * _F
_T = _M * (_M - 1) // 2
_L = 16

_ROWS, _COLS = np.triu_indices(_M, k=1)
_IDX = np.asarray(_ROWS * _M + _COLS, dtype=np.int32)


@jax.jit
def _triu_sc(x2, idx):
    info = plsc.get_sparse_core_info()
    nw = info.num_cores * info.num_subcores
    npw = _N // nw

    mesh = plsc.VectorSubcoreMesh(core_axis_name="c", subcore_axis_name="s")

    @functools.partial(
        pl.kernel,
        mesh=mesh,
        out_type=jax.ShapeDtypeStruct((_N, _T), jnp.float32),
        scratch_types=[
            pltpu.VMEM((_M * _M,), jnp.float32),
            pltpu.VMEM((_M * _M,), jnp.float32),
            pltpu.VMEM((_T,), jnp.float32),
            pltpu.VMEM((_T,), jnp.float32),
            pltpu.VMEM((_T,), jnp.int32),
            pltpu.SemaphoreType.DMA,
            pltpu.SemaphoreType.DMA,
        ],
        compiler_params=pltpu.CompilerParams(needs_layout_passes=False),
    )
    def k(x_hbm, idx_hbm, out_hbm, in0, in1, ou0, ou1, idxb, in_sem, out_sem):
        wid = lax.axis_index("s") * info.num_cores + lax.axis_index("c")
        base = wid * npw
        pltpu.sync_copy(idx_hbm, idxb)
        ins = (in0, in1)
        ous = (ou0, ou1)

        def gather(src, dst):
            def chunk(i, c):
                iv = idxb[pl.ds(i * _L, _L)]
                dst[pl.ds(i * _L, _L)] = plsc.load_gather(src, [iv])
                return c

            lax.fori_loop(0, _T // _L, chunk, 0, unroll=8)

        pltpu.async_copy(x_hbm.at[base], in0, in_sem)

        def outer(i, carry):
            n = base + 2 * i
            for b in range(2):
                m = n + b

                @pl.when(m + 1 < base + npw)
                def _():
                    pltpu.async_copy(x_hbm.at[m + 1], ins[1 - b], in_sem)

                pltpu.make_async_copy(x_hbm.at[m], ins[b], in_sem).wait()

                @pl.when(m - 2 >= base)
                def _():
                    pltpu.make_async_copy(
                        ous[b], out_hbm.at[m - 2], out_sem
                    ).wait()

                gather(ins[b], ous[b])
                pltpu.async_copy(ous[b], out_hbm.at[m], out_sem)
            return carry

        lax.fori_loop(0, npw // 2, outer, 0)
        pltpu.make_async_copy(ou0, out_hbm.at[base], out_sem).wait()
        pltpu.make_async_copy(ou1, out_hbm.at[base], out_sem).wait()

    return k(x2, idx)


def kernel(X):
    out = _triu_sc(X.reshape(_N, _M * _M), _IDX)
    return out.reshape(_B, _F, _T)


# speedup vs baseline: 1.2886x; 1.0029x over previous
import functools

import jax
import jax.numpy as jnp
import numpy as np
from jax import lax
from jax.experimental import pallas as pl
from jax.experimental.pallas import tpu as pltpu
from jax.experimental.pallas import tpu_sc as plsc

_B, _F, _M = 32, 64, 128
_N = _B * _F
_T = _M * (_M - 1) // 2
_L = 16

_R, _C = np.triu_indices(_M, k=1)
_ROWS = np.asarray(_R, dtype=np.int32)
_COLS = np.asarray(_C, dtype=np.int32)


@jax.jit
def _triu_sc(x3, rows, cols):
    info = plsc.get_sparse_core_info()
    nw = info.num_cores * info.num_subcores
    npw = _N // nw

    mesh = plsc.VectorSubcoreMesh(core_axis_name="c", subcore_axis_name="s")

    @functools.partial(
        pl.kernel,
        mesh=mesh,
        out_type=jax.ShapeDtypeStruct((_N, _T), jnp.float32),
        scratch_types=[
            pltpu.VMEM((_M, _M), jnp.float32),
            pltpu.VMEM((_M, _M), jnp.float32),
            pltpu.VMEM((_T,), jnp.float32),
            pltpu.VMEM((_T,), jnp.float32),
            pltpu.VMEM((_T,), jnp.int32),
            pltpu.VMEM((_T,), jnp.int32),
            pltpu.SemaphoreType.DMA,
            pltpu.SemaphoreType.DMA,
        ],
        compiler_params=pltpu.CompilerParams(needs_layout_passes=False),
    )
    def k(x_hbm, r_hbm, c_hbm, out_hbm, in0, in1, ou0, ou1, rb, cb,
          in_sem, out_sem):
        wid = lax.axis_index("s") * info.num_cores + lax.axis_index("c")
        base = wid * npw
        pltpu.sync_copy(r_hbm, rb)
        pltpu.sync_copy(c_hbm, cb)
        ins = (in0, in1)
        ous = (ou0, ou1)

        def gather(src, dst):
            def chunk(i, c):
                riv = rb[pl.ds(i * _L, _L)]
                civ = cb[pl.ds(i * _L, _L)]
                dst[pl.ds(i * _L, _L)] = plsc.load_gather(src, [riv, civ])
                return c

            lax.fori_loop(0, _T // _L, chunk, 0, unroll=8)

        pltpu.async_copy(x_hbm.at[base], in0, in_sem)

        def outer(i, carry):
            n = base + 2 * i
            for b in range(2):
                m = n + b

                @pl.when(m + 1 < base + npw)
                def _():
                    pltpu.async_copy(x_hbm.at[m + 1], ins[1 - b], in_sem)

                pltpu.make_async_copy(x_hbm.at[m], ins[b], in_sem).wait()

                @pl.when(m - 2 >= base)
                def _():
                    pltpu.make_async_copy(
                        ous[b], out_hbm.at[m - 2], out_sem
                    ).wait()

                gather(ins[b], ous[b])
                pltpu.async_copy(ous[b], out_hbm.at[m], out_sem)
            return carry

        lax.fori_loop(0, npw // 2, outer, 0)
        pltpu.make_async_copy(ou0, out_hbm.at[base], out_sem).wait()
        pltpu.make_async_copy(ou1, out_hbm.at[base], out_sem).wait()

    return k(x3, rows, cols)


def kernel(X):
    out = _triu_sc(X.reshape(_N, _M, _M), _ROWS, _COLS)
    return out.reshape(_B, _F, _T)


# speedup vs baseline: 2.4123x; 1.8721x over previous
import jax
import jax.numpy as jnp
import numpy as np
from jax.experimental import pallas as pl
from jax.experimental.pallas import tpu as pltpu

_B, _F, _M = 32, 64, 128
_N = _B * _F
_T = _M * (_M - 1) // 2
_G = 8

_LENS = [_M - 1 - r for r in range(_M - 1)]
_OFF = np.concatenate([[0], np.cumsum(_LENS)]).astype(np.int64)


def _tc_body(x_ref, o_ref):
    for r in range(_M - 1):
        L = _M - 1 - r
        o = int(_OFF[r])
        o_ref[:, o:o + L] = x_ref[:, r, r + 1:_M]


@jax.jit
def _triu_tc(x3):
    return pl.pallas_call(
        _tc_body,
        grid=(_N // _G,),
        in_specs=[
            pl.BlockSpec((_G, _M, _M), lambda i: (i, 0, 0)),
        ],
        out_specs=pl.BlockSpec((_G, _T), lambda i: (i, 0)),
        out_shape=jax.ShapeDtypeStruct((_N, _T), jnp.float32),
    )(x3)


def kernel(X):
    out = _triu_tc(X.reshape(_N, _M, _M))
    return out.reshape(_B, _F, _T)
